# Initial kernel scaffold; baseline (speedup 1.0000x reference)
import jax
import jax.numpy as jnp
from jax.experimental import pallas as pl


def kernel(embeddings, edge_index, sub_pos, sub_neg, raw_alpha):
    raise NotImplementedError("write your pallas kernel here")


# speedup vs baseline: 26.6985x; 26.6985x over previous
import functools

import jax
import jax.numpy as jnp
from jax import lax
from jax.experimental import pallas as pl
from jax.experimental.pallas import tpu as pltpu
from jax.experimental.pallas import tpu_sc as plsc

N = 10000
N_PAD = 10240
ROWS_PER_TILE = 640
EDGES = 320000
EPT = 20000
EJ = 157
EPT_PAD = EJ * 128
PAD_ROW = N + 200
K_STEPS = 5
NSUB = 1000
NSUB_PAD = 1024
EPS = 1e-6


def _build_sc_kernel():
    mesh = plsc.VectorSubcoreMesh(core_axis_name="c", subcore_axis_name="s")

    @functools.partial(
        pl.kernel,
        mesh=mesh,
        compiler_params=pltpu.CompilerParams(needs_layout_passes=False),
        out_type=[
            jax.ShapeDtypeStruct((N_PAD,), jnp.float32),
            jax.ShapeDtypeStruct((N_PAD,), jnp.float32),
            jax.ShapeDtypeStruct((2 * NSUB_PAD,), jnp.float32),
        ],
        scratch_types=[
            pltpu.VMEM((EJ, 128), jnp.int32),
            pltpu.VMEM((EJ, 128), jnp.int32),
            pltpu.VMEM((EJ, 128), jnp.float32),
            pltpu.VMEM((EJ, 128), jnp.float32),
            pltpu.VMEM((N_PAD,), jnp.float32),
            pltpu.VMEM((N_PAD,), jnp.float32),
            pltpu.VMEM((ROWS_PER_TILE,), jnp.float32),
            pltpu.VMEM((ROWS_PER_TILE,), jnp.float32),
            pltpu.VMEM((ROWS_PER_TILE,), jnp.float32),
            pltpu.VMEM((ROWS_PER_TILE,), jnp.float32),
            pltpu.VMEM((ROWS_PER_TILE,), jnp.float32),
            pltpu.VMEM((ROWS_PER_TILE,), jnp.float32),
            pltpu.VMEM((NSUB_PAD,), jnp.int32),
            pltpu.VMEM((NSUB_PAD,), jnp.int32),
            pltpu.VMEM((16,), jnp.float32),
            pltpu.VMEM((16,), jnp.float32),
            pltpu.VMEM_SHARED((N_PAD,), jnp.float32),
            pltpu.VMEM_SHARED((N_PAD,), jnp.float32),
            pltpu.VMEM_SHARED((N_PAD,), jnp.float32),
            pltpu.VMEM_SHARED((N_PAD,), jnp.float32),
        ],
    )
    def sc_kernel(rows_hbm, cols_hbm, pos_hbm, neg_hbm, alpha_hbm,
                  eout0, eout1, vals_out,
                  ridx, cidx, vals0, vals1, e_loc0, e_loc1,
                  accbuf0, accbuf1, enew0, enew1, dwrow, zbuf,
                  pos_v, neg_v, alpha_v, cbuf,
                  acc0_sh, acc1_sh, e0_sh, e1_sh):
        s = lax.axis_index("s")
        core = lax.axis_index("c")
        lane = lax.iota(jnp.int32, 16)
        zeros_f = jnp.zeros((16,), jnp.float32)
        ones_f = jnp.ones((16,), jnp.float32)
        row0 = s * ROWS_PER_TILE

        pltpu.sync_copy(rows_hbm.at[s], ridx)
        pltpu.sync_copy(cols_hbm.at[s], cidx)
        pltpu.sync_copy(pos_hbm, pos_v)
        pltpu.sync_copy(neg_hbm, neg_v)
        pltpu.sync_copy(alpha_hbm, alpha_v)
        araw = alpha_v[...]
        alpha = 1.0 / (1.0 + jnp.exp(-araw))
        one_m_alpha = 1.0 - alpha

        def zloop(k, carry):
            zbuf[pl.ds(k * 16, 16)] = zeros_f
            return carry
        lax.fori_loop(0, ROWS_PER_TILE // 16, zloop, 0)

        def eloop(k, carry):
            e_loc0[pl.ds(k * 16, 16)] = zeros_f
            e_loc1[pl.ds(k * 16, 16)] = zeros_f
            return carry
        lax.fori_loop(0, N_PAD // 16, eloop, 0)

        def sloop(k, carry):
            ip = pos_v[pl.ds(k * 16, 16)]
            plsc.store_scatter(e_loc1, [ip], ones_f)
            iq = neg_v[pl.ds(k * 16, 16)]
            plsc.store_scatter(e_loc0, [iq], ones_f)
            return carry
        lax.fori_loop(0, NSUB_PAD // 16, sloop, 0)

        pltpu.sync_copy(zbuf, acc0_sh.at[pl.ds(row0, ROWS_PER_TILE)])
        pltpu.sync_copy(zbuf, acc1_sh.at[pl.ds(row0, ROWS_PER_TILE)])
        plsc.subcore_barrier()

        def oloop(j, carry):
            for q in range(8):
                vals0[j, pl.ds(q * 16, 16)] = ones_f
            return carry
        lax.fori_loop(0, EJ, oloop, 0)

        def dscat(j, carry):
            pltpu.sync_copy(vals0.at[j], acc0_sh.at[ridx.at[j]], add=True)
            return carry
        lax.fori_loop(0, EJ, dscat, 0)
        plsc.subcore_barrier()

        pltpu.sync_copy(acc0_sh.at[pl.ds(row0, ROWS_PER_TILE)], accbuf0)

        def dloop(k, carry):
            deg = accbuf0[pl.ds(k * 16, 16)]
            dwrow[pl.ds(k * 16, 16)] = 1.0 / jnp.maximum(deg, 1e-12)
            return carry
        lax.fori_loop(0, ROWS_PER_TILE // 16, dloop, 0)
        pltpu.sync_copy(zbuf, acc0_sh.at[pl.ds(row0, ROWS_PER_TILE)])
        plsc.subcore_barrier()

        def step(step_i, carry):
            def gloop(j, c2):
                for q in range(8):
                    c = cidx[j, pl.ds(q * 16, 16)]
                    vals0[j, pl.ds(q * 16, 16)] = plsc.load_gather(e_loc0, [c])
                    vals1[j, pl.ds(q * 16, 16)] = plsc.load_gather(e_loc1, [c])
                return c2
            lax.fori_loop(0, EJ, gloop, 0)

            def scat(j, c2):
                pltpu.sync_copy(vals0.at[j], acc0_sh.at[ridx.at[j]], add=True)
                pltpu.sync_copy(vals1.at[j], acc1_sh.at[ridx.at[j]], add=True)
                return c2
            lax.fori_loop(0, EJ, scat, 0)
            plsc.subcore_barrier()

            pltpu.sync_copy(acc0_sh.at[pl.ds(row0, ROWS_PER_TILE)], accbuf0)
            pltpu.sync_copy(acc1_sh.at[pl.ds(row0, ROWS_PER_TILE)], accbuf1)

            def floop(k, c2):
                sl = pl.ds(k * 16, 16)
                n0 = accbuf0[sl]
                n1 = accbuf1[sl]
                d = dwrow[sl]
                e0 = e_loc0[pl.ds(row0 + k * 16, 16)]
                e1 = e_loc1[pl.ds(row0 + k * 16, 16)]
                enew0[sl] = alpha * e0 + one_m_alpha * (n0 * d)
                enew1[sl] = alpha * e1 + one_m_alpha * (n1 * d)
                return c2
            lax.fori_loop(0, ROWS_PER_TILE // 16, floop, 0)

            pltpu.sync_copy(zbuf, acc0_sh.at[pl.ds(row0, ROWS_PER_TILE)])
            pltpu.sync_copy(zbuf, acc1_sh.at[pl.ds(row0, ROWS_PER_TILE)])
            pltpu.sync_copy(enew0, e0_sh.at[pl.ds(row0, ROWS_PER_TILE)])
            pltpu.sync_copy(enew1, e1_sh.at[pl.ds(row0, ROWS_PER_TILE)])
            plsc.subcore_barrier()

            pltpu.sync_copy(e0_sh, e_loc0)
            pltpu.sync_copy(e1_sh, e_loc1)
            return carry
        lax.fori_loop(0, K_STEPS, step, 0)

        for q in range(4):
            base = (s * 4 + q) * 16
            msk = (base + lane) < NSUB
            ip = pos_v[pl.ds(base, 16)]
            gp = plsc.load_gather(e_loc1, [ip])
            cbuf[...] = jnp.where(msk, jnp.maximum(gp, EPS), 1.0)

            @pl.when(core == 0)
            def _():
                pltpu.sync_copy(cbuf, vals_out.at[pl.ds(base, 16)])

            iq = neg_v[pl.ds(base, 16)]
            gn = plsc.load_gather(e_loc0, [iq])
            cbuf[...] = jnp.where(msk, jnp.maximum(gn, EPS), 1.0)

            @pl.when(core == 0)
            def _():
                pltpu.sync_copy(cbuf, vals_out.at[pl.ds(NSUB_PAD + base, 16)])

        @pl.when(core == 0)
        def _():
            pltpu.sync_copy(e_loc0.at[pl.ds(row0, ROWS_PER_TILE)],
                            eout0.at[pl.ds(row0, ROWS_PER_TILE)])
            pltpu.sync_copy(e_loc1.at[pl.ds(row0, ROWS_PER_TILE)],
                            eout1.at[pl.ds(row0, ROWS_PER_TILE)])

    return sc_kernel


_SC_KERNEL = _build_sc_kernel()


def _loss_body(v_ref, o_ref):
    v = v_ref[...]
    total = -jnp.sum(jnp.log(v)) / jnp.float32(NSUB)
    o_ref[...] = jnp.full((8, 128), total, jnp.float32)


_LOSS_CALL = pl.pallas_call(
    _loss_body,
    out_shape=jax.ShapeDtypeStruct((8, 128), jnp.float32),
)


def kernel(embeddings, edge_index, sub_pos, sub_neg, raw_alpha):
    rows = edge_index[0]
    cols = edge_index[1]
    rows_p = jnp.concatenate(
        [rows.reshape(16, EPT),
         jnp.full((16, EPT_PAD - EPT), PAD_ROW, jnp.int32)],
        axis=1).reshape(16, EJ, 128)
    cols_p = jnp.concatenate(
        [cols.reshape(16, EPT),
         jnp.zeros((16, EPT_PAD - EPT), jnp.int32)],
        axis=1).reshape(16, EJ, 128)
    pos_p = jnp.concatenate(
        [sub_pos, jnp.full((NSUB_PAD - NSUB,), PAD_ROW, jnp.int32)])
    neg_p = jnp.concatenate(
        [sub_neg, jnp.full((NSUB_PAD - NSUB,), PAD_ROW, jnp.int32)])
    alpha16 = jnp.full((16,), raw_alpha, jnp.float32)

    e0, e1, vals = _SC_KERNEL(rows_p, cols_p, pos_p, neg_p, alpha16)
    E = jnp.stack([e0[:N], e1[:N]], axis=1)
    lp = _LOSS_CALL(vals.reshape(16, 128))[0, 0]
    return (lp, E)
